# baseline (speedup 1.0000x reference)
import jax
import jax.numpy as jnp
from jax.experimental import pallas as pl
from jax.experimental.pallas import tpu as pltpu

_TOKENS = 32768
_HIDDEN = 4096
_EXPERTS = 64
_BT = 512


def _softmax_argmax(logits, probs_ref, sel_ref, half):
    m = jnp.max(logits, axis=-1, keepdims=True)
    e = jnp.exp(logits - m)
    probs = e / jnp.sum(e, axis=-1, keepdims=True)
    probs_ref[half] = probs
    pm = jnp.max(probs, axis=-1, keepdims=True)
    idx = jax.lax.broadcasted_iota(jnp.int32, probs.shape, 1)
    sel = jnp.min(jnp.where(probs == pm, idx, _EXPERTS), axis=-1)
    sel_ref[half, 0, 0, :] = sel


def _router_block(hs_a, hs_b, w_ref, probs_ref, sel_ref):
    w = w_ref[:]
    la = jax.lax.dot_general(hs_a[0], w, (((1,), (1,)), ((), ())),
                             preferred_element_type=jnp.float32)
    lb = jax.lax.dot_general(hs_b[0], w, (((1,), (1,)), ((), ())),
                             preferred_element_type=jnp.float32)
    _softmax_argmax(la, probs_ref, sel_ref, 0)
    _softmax_argmax(lb, probs_ref, sel_ref, 1)


def kernel(hidden_states, W):
    nb2 = _TOKENS // _BT // 2
    hs3 = hidden_states.reshape(2, _TOKENS // 2, _HIDDEN)
    probs, sel = pl.pallas_call(
        _router_block,
        grid=(nb2,),
        in_specs=[
            pl.BlockSpec((1, _BT, _HIDDEN), lambda i: (0, i, 0)),
            pl.BlockSpec((1, _BT, _HIDDEN), lambda i: (1, i, 0)),
            pl.BlockSpec((_EXPERTS, _HIDDEN), lambda i: (0, 0)),
        ],
        out_specs=[
            pl.BlockSpec((2, _BT, _EXPERTS), lambda i: (0, i, 0)),
            pl.BlockSpec((2, 1, 1, _BT), lambda i: (0, i, 0, 0)),
        ],
        out_shape=[
            jax.ShapeDtypeStruct((2, _TOKENS // 2, _EXPERTS), jnp.float32),
            jax.ShapeDtypeStruct((2, nb2, 1, _BT), jnp.int32),
        ],
        compiler_params=pltpu.CompilerParams(
            dimension_semantics=("parallel",),
        ),
    )(hs3, hs3, W)
    return probs.reshape(_TOKENS, _EXPERTS), sel.reshape(_TOKENS)


# speedup vs baseline: 1.0496x; 1.0496x over previous
import jax
import jax.numpy as jnp
from jax.experimental import pallas as pl
from jax.experimental.pallas import tpu as pltpu

_TOKENS = 32768
_HIDDEN = 4096
_EXPERTS = 64
_CH = 512
_NCH = _TOKENS // _CH
_NBUF = 5


def _router_body(hs_hbm, w_ref, probs_ref, sel_ref, buf, sems):
    def chunk_copy(i, slot):
        return pltpu.make_async_copy(
            hs_hbm.at[pl.ds(i * _CH, _CH), :], buf.at[slot], sems.at[slot])

    for k in range(_NBUF - 1):
        chunk_copy(k, k).start()

    def step(i, carry):
        slot = jax.lax.rem(i, _NBUF)
        chunk_copy(i, slot).wait()

        nxt = i + _NBUF - 1
        @pl.when(nxt < _NCH)
        def _():
            chunk_copy(nxt, jax.lax.rem(nxt, _NBUF)).start()

        logits = jax.lax.dot_general(
            buf[slot], w_ref[:], (((1,), (1,)), ((), ())),
            preferred_element_type=jnp.float32)
        m = jnp.max(logits, axis=-1, keepdims=True)
        e = jnp.exp(logits - m)
        probs = e / jnp.sum(e, axis=-1, keepdims=True)
        probs_ref[pl.ds(i * _CH, _CH), :] = probs
        pm = jnp.max(probs, axis=-1, keepdims=True)
        idx = jax.lax.broadcasted_iota(jnp.int32, probs.shape, 1)
        sel = jnp.min(jnp.where(probs == pm, idx, _EXPERTS), axis=-1)
        sel_ref[i, :] = sel
        return carry

    jax.lax.fori_loop(0, _NCH, step, 0)


def kernel(hidden_states, W):
    probs, sel = pl.pallas_call(
        _router_body,
        in_specs=[
            pl.BlockSpec(memory_space=pltpu.HBM),
            pl.BlockSpec(memory_space=pltpu.VMEM),
        ],
        out_specs=[
            pl.BlockSpec(memory_space=pltpu.VMEM),
            pl.BlockSpec(memory_space=pltpu.VMEM),
        ],
        out_shape=[
            jax.ShapeDtypeStruct((_TOKENS, _EXPERTS), jnp.float32),
            jax.ShapeDtypeStruct((_NCH, _CH), jnp.int32),
        ],
        scratch_shapes=[
            pltpu.VMEM((_NBUF, _CH, _HIDDEN), jnp.float32),
            pltpu.SemaphoreType.DMA((_NBUF,)),
        ],
        compiler_params=pltpu.CompilerParams(
            vmem_limit_bytes=100 * 1024 * 1024,
        ),
    )(hidden_states, W)
    return probs, sel.reshape(_TOKENS)
